# baseline (speedup 1.0000x reference)
import functools

import jax
import jax.numpy as jnp
from jax.experimental import pallas as pl
from jax.experimental.pallas import tpu as pltpu

RATE = 0.7
ALPHA = 0.8
BETA = 1.2

NBUF = 4


def _fused_kernel(x_hbm, o_ref, acc_ref, buf_ref, sem_ref, *, n_chunks, cc, k):
    b = x_hbm.shape[0]
    bi = pl.program_id(0)
    total = b * n_chunks

    def copy(g, slot):
        return pltpu.make_async_copy(
            x_hbm.at[g // n_chunks, pl.ds(jax.lax.rem(g, n_chunks) * cc, cc)],
            buf_ref.at[slot],
            sem_ref.at[slot],
        )

    @pl.when(bi == 0)
    def _prime():
        for s in range(min(NBUF, total)):
            copy(s, s).start()

    acc_ref[...] = jnp.zeros_like(acc_ref)

    def body(j, carry):
        g = bi * n_chunks + j
        slot = jax.lax.rem(g, NBUF)
        copy(g, slot).wait()
        acc_ref[...] += jnp.sum(buf_ref[slot], axis=0)
        nxt = g + NBUF

        @pl.when(nxt < total)
        def _refill():
            copy(nxt, slot).start()

        return carry

    jax.lax.fori_loop(0, n_chunks, body, 0)

    h = acc_ref[...]
    i32 = jax.lax.bitcast_convert_type(h, jnp.int32)
    key = jnp.where(i32 < 0, i32 ^ 0x7FFFFFFF, i32)
    ukey = jax.lax.bitcast_convert_type(key, jnp.uint32) ^ jnp.uint32(0x80000000)

    def sbody(t, T):
        bit = jnp.uint32(31 - t)
        cand = T | (jnp.uint32(1) << bit)
        cnt = jnp.sum((ukey >= cand).astype(jnp.int32))
        return jnp.where(cnt >= k, cand, T)

    T = jax.lax.fori_loop(0, 32, sbody, jnp.uint32(0))

    kk = jax.lax.bitcast_convert_type(T ^ jnp.uint32(0x80000000), jnp.int32)
    iv = jnp.where(kk < 0, kk ^ 0x7FFFFFFF, kk)
    v = jax.lax.bitcast_convert_type(iv, jnp.float32)
    o_ref[0] = jnp.where(h >= v, jnp.float32(ALPHA), jnp.float32(BETA))


def kernel(inputs):
    b, c, h, w = inputs.shape
    hw = h * w
    lanes = 128
    rows = hw // lanes
    k = int(RATE * hw)
    cc = 32
    n_chunks = c // cc
    x = inputs.reshape(b, c, rows, lanes)
    out = pl.pallas_call(
        functools.partial(_fused_kernel, n_chunks=n_chunks, cc=cc, k=k),
        grid=(b,),
        in_specs=[pl.BlockSpec(memory_space=pltpu.HBM)],
        out_specs=pl.BlockSpec((1, rows, lanes), lambda i: (i, 0, 0)),
        out_shape=jax.ShapeDtypeStruct((b, rows, lanes), jnp.float32),
        scratch_shapes=[
            pltpu.VMEM((rows, lanes), jnp.float32),
            pltpu.VMEM((NBUF, cc, rows, lanes), jnp.float32),
            pltpu.SemaphoreType.DMA((NBUF,)),
        ],
    )(x)
    return out.reshape(b, 1, h, w)


# speedup vs baseline: 1.0011x; 1.0011x over previous
import functools

import jax
import jax.numpy as jnp
from jax.experimental import pallas as pl
from jax.experimental.pallas import tpu as pltpu

RATE = 0.7
ALPHA = 0.8
BETA = 1.2

NBUF = 8


def _fused_kernel(x_hbm, o_ref, acc_ref, buf_ref, sem_ref, *, n_chunks, cc, k):
    b = x_hbm.shape[0]
    bi = pl.program_id(0)
    total = b * n_chunks

    def copy(g, slot):
        return pltpu.make_async_copy(
            x_hbm.at[g // n_chunks, pl.ds(jax.lax.rem(g, n_chunks) * cc, cc)],
            buf_ref.at[slot],
            sem_ref.at[slot],
        )

    @pl.when(bi == 0)
    def _prime():
        for s in range(min(NBUF, total)):
            copy(s, s).start()

    acc_ref[...] = jnp.zeros_like(acc_ref)

    def body(j, carry):
        g = bi * n_chunks + j
        slot = jax.lax.rem(g, NBUF)
        copy(g, slot).wait()
        acc_ref[...] += jnp.sum(buf_ref[slot], axis=0)
        nxt = g + NBUF

        @pl.when(nxt < total)
        def _refill():
            copy(nxt, slot).start()

        return carry

    jax.lax.fori_loop(0, n_chunks, body, 0)

    h = acc_ref[...]
    i32 = jax.lax.bitcast_convert_type(h, jnp.int32)
    key = jnp.where(i32 < 0, i32 ^ 0x7FFFFFFF, i32)
    ukey = jax.lax.bitcast_convert_type(key, jnp.uint32) ^ jnp.uint32(0x80000000)

    def sbody(t, T):
        bit = jnp.uint32(31 - t)
        cand = T | (jnp.uint32(1) << bit)
        cnt = jnp.sum((ukey >= cand).astype(jnp.int32))
        return jnp.where(cnt >= k, cand, T)

    T = jax.lax.fori_loop(0, 32, sbody, jnp.uint32(0))

    kk = jax.lax.bitcast_convert_type(T ^ jnp.uint32(0x80000000), jnp.int32)
    iv = jnp.where(kk < 0, kk ^ 0x7FFFFFFF, kk)
    v = jax.lax.bitcast_convert_type(iv, jnp.float32)
    o_ref[0] = jnp.where(h >= v, jnp.float32(ALPHA), jnp.float32(BETA))


def kernel(inputs):
    b, c, h, w = inputs.shape
    hw = h * w
    lanes = 128
    rows = hw // lanes
    k = int(RATE * hw)
    cc = 16
    n_chunks = c // cc
    x = inputs.reshape(b, c, rows, lanes)
    out = pl.pallas_call(
        functools.partial(_fused_kernel, n_chunks=n_chunks, cc=cc, k=k),
        grid=(b,),
        in_specs=[pl.BlockSpec(memory_space=pltpu.HBM)],
        out_specs=pl.BlockSpec((1, rows, lanes), lambda i: (i, 0, 0)),
        out_shape=jax.ShapeDtypeStruct((b, rows, lanes), jnp.float32),
        scratch_shapes=[
            pltpu.VMEM((rows, lanes), jnp.float32),
            pltpu.VMEM((NBUF, cc, rows, lanes), jnp.float32),
            pltpu.SemaphoreType.DMA((NBUF,)),
        ],
    )(x)
    return out.reshape(b, 1, h, w)
